# Initial kernel scaffold; baseline (speedup 1.0000x reference)
import jax
import jax.numpy as jnp
from jax.experimental import pallas as pl


def kernel(prediction, target):
    raise NotImplementedError("write your pallas kernel here")


# speedup vs baseline: 2.2745x; 2.2745x over previous
import jax
import jax.numpy as jnp
from jax.experimental import pallas as pl
from jax.experimental.pallas import tpu as pltpu

_NUM_CLASSES = 20
_A = 5
_ANCHORS = (
    (1.3221, 1.73145),
    (3.19275, 4.00944),
    (5.05587, 8.09892),
    (9.47112, 4.84053),
    (11.2364, 10.0071),
)
_LAMBDA_COORD = 5.0
_LAMBDA_OBJ = 1.0
_LAMBDA_NOOBJ = 0.5
_LAMBDA_CLS = 1.0


def _loss_body(pred_ref, tgt_ref, out_ref):
    b = pl.program_id(0)
    p = pred_ref[0]
    t = tgt_ref[0]

    gt_conf = t[20]
    gx, gy, gw, gh = t[21], t[22], t[23], t[24]
    obj = (gt_conf != 0.0).astype(jnp.float32)
    noobj = ((1.0 - gt_conf) != 0.0).astype(jnp.float32)

    bx1 = gx - gw / 2
    by1 = gy - gh / 2
    bx2 = gx + gw / 2
    by2 = gy + gh / 2
    area_b = (bx2 - bx1) * (by2 - by1)

    px_l = []
    py_l = []
    pw_l = []
    ph_l = []
    pc_l = []
    iou_l = []
    for a in range(_A):
        base = 25 * a
        x = jax.nn.sigmoid(p[base + 21])
        y = jax.nn.sigmoid(p[base + 22])
        w = jnp.exp(p[base + 23]) * _ANCHORS[a][0]
        h = jnp.exp(p[base + 24]) * _ANCHORS[a][1]
        c = jax.nn.sigmoid(p[base + 20])
        ax1 = x - w / 2
        ay1 = y - h / 2
        ax2 = x + w / 2
        ay2 = y + h / 2
        iw = jnp.clip(jnp.minimum(ax2, bx2) - jnp.maximum(ax1, bx1), 0.0, None)
        ih = jnp.clip(jnp.minimum(ay2, by2) - jnp.maximum(ay1, by1), 0.0, None)
        inter = iw * ih
        area_a = (ax2 - ax1) * (ay2 - ay1)
        iou = inter / (area_a + area_b - inter + 1e-9)
        px_l.append(x)
        py_l.append(y)
        pw_l.append(w)
        ph_l.append(h)
        pc_l.append(c)
        iou_l.append(iou)

    best_iou = iou_l[0]
    best = jnp.zeros_like(best_iou, dtype=jnp.int32)
    for a in range(1, _A):
        m = iou_l[a] > best_iou
        best_iou = jnp.where(m, iou_l[a], best_iou)
        best = jnp.where(m, a, best)

    sel = [(best == a).astype(jnp.float32) for a in range(_A)]
    bx = sum(sel[a] * px_l[a] for a in range(_A))
    by = sum(sel[a] * py_l[a] for a in range(_A))
    bw = sum(sel[a] * pw_l[a] for a in range(_A))
    bh = sum(sel[a] * ph_l[a] for a in range(_A))
    pc = sum(sel[a] * pc_l[a] for a in range(_A))

    box_term = obj * ((bx - gx) ** 2 + (by - gy) ** 2 + (bw - gw) ** 2 + (bh - gh) ** 2)
    conf_term = obj * (pc - gt_conf) ** 2
    noobj_term = noobj * pc * pc

    cls_rows = [sum(sel[a] * p[25 * a + j] for a in range(_A)) for j in range(_NUM_CLASSES)]
    mx = cls_rows[0]
    for j in range(1, _NUM_CLASSES):
        mx = jnp.maximum(mx, cls_rows[j])
    se = sum(jnp.exp(cls_rows[j] - mx) for j in range(_NUM_CLASSES))
    lse = mx + jnp.log(se)

    g_best_v = t[0]
    g_best = jnp.zeros_like(g_best_v, dtype=jnp.int32)
    for j in range(1, _NUM_CLASSES):
        m = t[j] > g_best_v
        g_best_v = jnp.where(m, t[j], g_best_v)
        g_best = jnp.where(m, j, g_best)
    picked = sum((g_best == j).astype(jnp.float32) * cls_rows[j] for j in range(_NUM_CLASSES))
    cls_term = obj * (lse - picked)

    vals = jnp.stack(
        [jnp.sum(box_term), jnp.sum(conf_term), jnp.sum(noobj_term), jnp.sum(cls_term)]
    ).reshape(1, 4)

    @pl.when(b == 0)
    def _():
        out_ref[...] = vals

    @pl.when(b != 0)
    def _():
        out_ref[...] += vals


def kernel(prediction, target):
    bsize, _, h, w = prediction.shape
    npix = h * w
    pred = prediction.reshape(bsize, 125, npix)
    tgt = jnp.transpose(target.reshape(bsize, npix, 25), (0, 2, 1))

    out = pl.pallas_call(
        _loss_body,
        grid=(bsize,),
        in_specs=[
            pl.BlockSpec((1, 125, npix), lambda b: (b, 0, 0)),
            pl.BlockSpec((1, 25, npix), lambda b: (b, 0, 0)),
        ],
        out_specs=pl.BlockSpec((1, 4), lambda b: (0, 0)),
        out_shape=jax.ShapeDtypeStruct((1, 4), jnp.float32),
        compiler_params=pltpu.CompilerParams(
            dimension_semantics=("arbitrary",),
        ),
    )(pred, tgt)

    inv_b = 1.0 / bsize
    box_loss = out[0, 0] * (inv_b * _LAMBDA_COORD)
    conf_loss = out[0, 1] * (inv_b * _LAMBDA_OBJ)
    noobj_loss = out[0, 2] * (inv_b * _LAMBDA_NOOBJ)
    cls_loss = out[0, 3] * (inv_b * _LAMBDA_CLS)
    return (box_loss, conf_loss, noobj_loss, cls_loss)


# speedup vs baseline: 3.5022x; 1.5398x over previous
import jax
import jax.numpy as jnp
from jax.experimental import pallas as pl
from jax.experimental.pallas import tpu as pltpu

_NUM_CLASSES = 20
_A = 5
_ANCHOR_W = (1.3221, 3.19275, 5.05587, 9.47112, 11.2364)
_ANCHOR_H = (1.73145, 4.00944, 8.09892, 4.84053, 10.0071)
_LAMBDA_COORD = 5.0
_LAMBDA_OBJ = 1.0
_LAMBDA_NOOBJ = 0.5
_LAMBDA_CLS = 1.0


def _loss_body(pred_ref, tgt_ref, out_ref):
    b = pl.program_id(0)
    p = pred_ref[0]
    t = tgt_ref[0]

    gt_conf = t[20:21]
    gx, gy, gw, gh = t[21:22], t[22:23], t[23:24], t[24:25]
    obj = (gt_conf != 0.0).astype(jnp.float32)
    noobj = ((1.0 - gt_conf) != 0.0).astype(jnp.float32)

    bx1 = gx - gw / 2
    by1 = gy - gh / 2
    bx2 = gx + gw / 2
    by2 = gy + gh / 2
    area_b = (bx2 - bx1) * (by2 - by1)

    def field(f):
        return jnp.concatenate([p[25 * a + f : 25 * a + f + 1] for a in range(_A)], axis=0)

    conf = jax.nn.sigmoid(field(20))
    x = jax.nn.sigmoid(field(21))
    y = jax.nn.sigmoid(field(22))
    w = jnp.concatenate(
        [jnp.exp(p[25 * a + 23 : 25 * a + 24]) * _ANCHOR_W[a] for a in range(_A)], axis=0
    )
    h = jnp.concatenate(
        [jnp.exp(p[25 * a + 24 : 25 * a + 25]) * _ANCHOR_H[a] for a in range(_A)], axis=0
    )

    ax1 = x - w / 2
    ay1 = y - h / 2
    ax2 = x + w / 2
    ay2 = y + h / 2
    iw = jnp.clip(jnp.minimum(ax2, bx2) - jnp.maximum(ax1, bx1), 0.0, None)
    ih = jnp.clip(jnp.minimum(ay2, by2) - jnp.maximum(ay1, by1), 0.0, None)
    inter = iw * ih
    area_a = (ax2 - ax1) * (ay2 - ay1)
    iou = inter / (area_a + area_b - inter + 1e-9)

    best = jnp.argmax(iou, axis=0)[None, :]
    sel = (jax.lax.broadcasted_iota(jnp.int32, iou.shape, 0) == best).astype(jnp.float32)

    bx = jnp.sum(sel * x, axis=0, keepdims=True)
    by = jnp.sum(sel * y, axis=0, keepdims=True)
    bw = jnp.sum(sel * w, axis=0, keepdims=True)
    bh = jnp.sum(sel * h, axis=0, keepdims=True)
    pc = jnp.sum(sel * conf, axis=0, keepdims=True)

    box_term = obj * ((bx - gx) ** 2 + (by - gy) ** 2 + (bw - gw) ** 2 + (bh - gh) ** 2)
    conf_term = obj * (pc - gt_conf) ** 2
    noobj_term = noobj * pc * pc

    cls_sel = sel[0:1] * p[0:_NUM_CLASSES]
    for a in range(1, _A):
        cls_sel = cls_sel + sel[a : a + 1] * p[25 * a : 25 * a + _NUM_CLASSES]

    mx = jnp.max(cls_sel, axis=0, keepdims=True)
    se = jnp.sum(jnp.exp(cls_sel - mx), axis=0, keepdims=True)
    lse = mx + jnp.log(se)

    g = t[0:_NUM_CLASSES]
    g_best = jnp.argmax(g, axis=0)[None, :]
    onehot = (jax.lax.broadcasted_iota(jnp.int32, g.shape, 0) == g_best).astype(jnp.float32)
    picked = jnp.sum(onehot * cls_sel, axis=0, keepdims=True)
    cls_term = obj * (lse - picked)

    vals = jnp.stack(
        [jnp.sum(box_term), jnp.sum(conf_term), jnp.sum(noobj_term), jnp.sum(cls_term)]
    ).reshape(1, 4)

    @pl.when(b == 0)
    def _():
        out_ref[...] = vals

    @pl.when(b != 0)
    def _():
        out_ref[...] += vals


def kernel(prediction, target):
    bsize, _, h, w = prediction.shape
    npix = h * w
    pred = prediction.reshape(bsize, 125, npix)
    tgt = jnp.transpose(target.reshape(bsize, npix, 25), (0, 2, 1))

    out = pl.pallas_call(
        _loss_body,
        grid=(bsize,),
        in_specs=[
            pl.BlockSpec((1, 125, npix), lambda b: (b, 0, 0)),
            pl.BlockSpec((1, 25, npix), lambda b: (b, 0, 0)),
        ],
        out_specs=pl.BlockSpec((1, 4), lambda b: (0, 0)),
        out_shape=jax.ShapeDtypeStruct((1, 4), jnp.float32),
        compiler_params=pltpu.CompilerParams(
            dimension_semantics=("arbitrary",),
        ),
    )(pred, tgt)

    inv_b = 1.0 / bsize
    box_loss = out[0, 0] * (inv_b * _LAMBDA_COORD)
    conf_loss = out[0, 1] * (inv_b * _LAMBDA_OBJ)
    noobj_loss = out[0, 2] * (inv_b * _LAMBDA_NOOBJ)
    cls_loss = out[0, 3] * (inv_b * _LAMBDA_CLS)
    return (box_loss, conf_loss, noobj_loss, cls_loss)
